# Initial kernel scaffold; baseline (speedup 1.0000x reference)
import jax
import jax.numpy as jnp
from jax.experimental import pallas as pl


def kernel(xs_meta, pairs_meta, params):
    raise NotImplementedError("write your pallas kernel here")


# speedup vs baseline: 664.7281x; 664.7281x over previous
import functools

import jax
import jax.numpy as jnp
from jax.experimental import pallas as pl

_BS = 8
_NUM_ROWS = 64
_NUM_COLS = 64
_NPAIRS = 16
_POS_ENC_DIM = 15
_SET_H = 64
_WH = 64
_OUT_SIZES = [(16, 128, 8), (128, 128, 8), (128, 64, 8)]
_NUM_CLASSES = 2
_GAT_OUT = 64


def _lin(x, w, b):
    return x @ w.T + b


def _d2v_batch(pairs_meta, p):
    x = jax.nn.relu(_lin(pairs_meta, p["f1_w"], p["f1_b"]))
    x = x + jax.nn.relu(_lin(x, p["f2r_w"], p["f2r_b"]))
    x = jax.nn.relu(_lin(x, p["f5_w"], p["f5_b"]))
    x = jnp.mean(x, axis=1)
    x_save = x
    x = jax.nn.relu(_lin(x, p["g1_w"], p["g1_b"]))
    x = jax.nn.relu(_lin(x, p["g2_w"], p["g2_b"]))
    x = jnp.mean(x, axis=1)
    x = jax.nn.relu(_lin(x, p["h1_w"], p["h1_b"]))
    x = x + jax.nn.relu(_lin(x, p["h2r_w"], p["h2r_b"]))
    x = jax.nn.relu(_lin(x, p["h5_w"], p["h5_b"]))
    pos = _lin(jax.nn.relu(_lin(x_save, p["p1_w"], p["p1_b"])),
               p["p2_w"], p["p2_b"])
    return x, pos


def _norm_rows(v):
    n = jnp.sqrt(jnp.sum(v * v, axis=-1, keepdims=True))
    return v / jnp.maximum(n, 1e-12)


def _weight_gen_batch(embed, p):
    out = []
    for i, (gin, godim, hds) in enumerate(_OUT_SIZES):
        gout = godim // hds
        all_w = _lin(jax.nn.relu(_lin(embed, p[f"wg{i}_w1"], p[f"wg{i}_b1"])),
                     p[f"wg{i}_w2"], p[f"wg{i}_b2"])
        s0 = gout * hds * gin
        s1 = hds * gout
        lin_w = _norm_rows(all_w[:, :s0]).reshape(_BS, gout * hds, gin)
        src_w = _norm_rows(all_w[:, s0:s0 + s1]).reshape(_BS, 1, hds * gout)
        dst_w = _norm_rows(all_w[:, s0 + s1:s0 + 2 * s1]).reshape(_BS, 1, hds * gout)
        bias_w = _norm_rows(all_w[:, s0 + 2 * s1:]).reshape(_BS, 1, gout * hds)
        out.append((lin_w, src_w, dst_w, bias_w))
    lw = _lin(jax.nn.relu(_lin(embed, p["wgl_w1"], p["wgl_b1"])),
              p["wgl_w2"], p["wgl_b2"]).reshape(_BS, _NUM_CLASSES, _GAT_OUT)
    return out, lw


def _gat_layer(x, w, srcv, dstv, bias, hds, gout):
    godim = hds * gout
    n = x.shape[0]
    xl = jax.lax.dot_general(x, w, (((1,), (1,)), ((), ())),
                             preferred_element_type=jnp.float32)
    cidx = jax.lax.broadcasted_iota(jnp.int32, (hds, godim), 1)
    hidx = jax.lax.broadcasted_iota(jnp.int32, (hds, godim), 0)
    mh = (cidx // gout == hidx).astype(jnp.float32)
    a_src_t = jax.lax.dot_general(mh * srcv, xl, (((1,), (1,)), ((), ())),
                                  preferred_element_type=jnp.float32)
    a_dst = jax.lax.dot_general(xl * dstv, mh, (((1,), (1,)), ((), ())),
                                preferred_element_type=jnp.float32)
    att_rows = []
    for h in range(hds):
        s = a_dst[:, h:h + 1] + a_src_t[h:h + 1, :]
        s = jnp.where(s >= 0, s, 0.2 * s)
        m = jnp.max(s, axis=1, keepdims=True)
        ex = jnp.exp(s - m)
        den = jnp.sum(ex, axis=1, keepdims=True) + 1e-16
        att_rows.append(ex / den)
    att = jnp.concatenate(att_rows, axis=0)
    out2 = jax.lax.dot_general(att, xl, (((1,), (0,)), ((), ())),
                               preferred_element_type=jnp.float32)
    out = bias * jnp.ones((n, 1), jnp.float32)
    for h in range(hds):
        out = out + out2[h * n:(h + 1) * n, :] * mh[h:h + 1, :]
    return out


def _gnn_kernel(x_ref, w0, s0, d0, b0, w1, s1, d1, b1, w2, s2, d2, b2,
                lw_ref, out_ref):
    x = x_ref[0, 0]
    x = _gat_layer(x, w0[0], s0[0], d0[0], b0[0], 8, 16)
    x = _gat_layer(x, w1[0], s1[0], d1[0], b1[0], 8, 16)
    x = _gat_layer(x, w2[0], s2[0], d2[0], b2[0], 8, 8)
    cols = jnp.sum(x, axis=0, keepdims=True)
    pred = jax.lax.dot_general(cols, lw_ref[0], (((1,), (1,)), ((), ())),
                               preferred_element_type=jnp.float32)
    out_ref[0, 0] = pred


def _bcast_spec(shape):
    nd = len(shape)
    return pl.BlockSpec((1,) + shape[1:],
                        lambda b, r, _nd=nd: (b,) + (0,) * (_nd - 1))


@jax.jit
def kernel(xs_meta, pairs_meta, params):
    embed, pos_enc = _d2v_batch(pairs_meta, params)
    gat_w, lin_w = _weight_gen_batch(embed, params)
    (w0, s0, d0, b0), (w1, s1, d1, b1), (w2, s2, d2, b2) = gat_w

    xs4 = xs_meta.reshape(_BS, _NUM_ROWS, _NUM_COLS, 1)
    pe = jnp.broadcast_to(pos_enc[:, None, :, :],
                          (_BS, _NUM_ROWS, _NUM_COLS, _POS_ENC_DIM))
    x_all = jnp.concatenate([xs4, pe], axis=-1)

    grid = (_BS, _NUM_ROWS)
    out = pl.pallas_call(
        _gnn_kernel,
        grid=grid,
        in_specs=[
            pl.BlockSpec((1, 1, _NUM_COLS, 16), lambda b, r: (b, r, 0, 0)),
            _bcast_spec(w0.shape), _bcast_spec(s0.shape),
            _bcast_spec(d0.shape), _bcast_spec(b0.shape),
            _bcast_spec(w1.shape), _bcast_spec(s1.shape),
            _bcast_spec(d1.shape), _bcast_spec(b1.shape),
            _bcast_spec(w2.shape), _bcast_spec(s2.shape),
            _bcast_spec(d2.shape), _bcast_spec(b2.shape),
            _bcast_spec(lin_w.shape),
        ],
        out_specs=pl.BlockSpec((1, 1, 1, _NUM_CLASSES),
                               lambda b, r: (b, r, 0, 0)),
        out_shape=jax.ShapeDtypeStruct((_BS, _NUM_ROWS, 1, _NUM_CLASSES),
                                       jnp.float32),
    )(x_all, w0, s0, d0, b0, w1, s1, d1, b1, w2, s2, d2, b2, lin_w)
    return out.reshape(_BS, _NUM_ROWS, _NUM_CLASSES)


# speedup vs baseline: 1336.2692x; 2.0102x over previous
import jax
import jax.numpy as jnp
from jax.experimental import pallas as pl

_BS = 8
_NUM_ROWS = 64
_NUM_COLS = 64
_POS_ENC_DIM = 15
_OUT_SIZES = [(16, 128, 8), (128, 128, 8), (128, 64, 8)]
_NUM_CLASSES = 2
_GAT_OUT = 64
_ROWS_PER = 8
_NB = _ROWS_PER * _NUM_COLS


def _lin(x, w, b):
    return x @ w.T + b


def _d2v_batch(pairs_meta, p):
    x = jax.nn.relu(_lin(pairs_meta, p["f1_w"], p["f1_b"]))
    x = x + jax.nn.relu(_lin(x, p["f2r_w"], p["f2r_b"]))
    x = jax.nn.relu(_lin(x, p["f5_w"], p["f5_b"]))
    x = jnp.mean(x, axis=1)
    x_save = x
    x = jax.nn.relu(_lin(x, p["g1_w"], p["g1_b"]))
    x = jax.nn.relu(_lin(x, p["g2_w"], p["g2_b"]))
    x = jnp.mean(x, axis=1)
    x = jax.nn.relu(_lin(x, p["h1_w"], p["h1_b"]))
    x = x + jax.nn.relu(_lin(x, p["h2r_w"], p["h2r_b"]))
    x = jax.nn.relu(_lin(x, p["h5_w"], p["h5_b"]))
    pos = _lin(jax.nn.relu(_lin(x_save, p["p1_w"], p["p1_b"])),
               p["p2_w"], p["p2_b"])
    return x, pos


def _norm_rows(v):
    n = jnp.sqrt(jnp.sum(v * v, axis=-1, keepdims=True))
    return v / jnp.maximum(n, 1e-12)


def _weight_gen_batch(embed, p):
    out = []
    for i, (gin, godim, hds) in enumerate(_OUT_SIZES):
        gout = godim // hds
        all_w = _lin(jax.nn.relu(_lin(embed, p[f"wg{i}_w1"], p[f"wg{i}_b1"])),
                     p[f"wg{i}_w2"], p[f"wg{i}_b2"])
        s0 = gout * hds * gin
        s1 = hds * gout
        lin_w = _norm_rows(all_w[:, :s0]).reshape(_BS, gout * hds, gin)
        src_w = _norm_rows(all_w[:, s0:s0 + s1]).reshape(_BS, 1, hds * gout)
        dst_w = _norm_rows(all_w[:, s0 + s1:s0 + 2 * s1]).reshape(_BS, 1, hds * gout)
        bias_w = _norm_rows(all_w[:, s0 + 2 * s1:]).reshape(_BS, 1, gout * hds)
        out.append((lin_w, src_w, dst_w, bias_w))
    lw = _lin(jax.nn.relu(_lin(embed, p["wgl_w1"], p["wgl_b1"])),
              p["wgl_w2"], p["wgl_b2"]).reshape(_BS, _NUM_CLASSES, _GAT_OUT)
    return out, lw


def _gat_layer(x, w, srcv, dstv, bias, hds, gout):
    godim = hds * gout
    n = _NUM_COLS
    xl = jax.lax.dot_general(x, w, (((1,), (1,)), ((), ())),
                             preferred_element_type=jnp.float32)
    cidx = jax.lax.broadcasted_iota(jnp.int32, (hds, godim), 1)
    hidx = jax.lax.broadcasted_iota(jnp.int32, (hds, godim), 0)
    mh = (cidx // gout == hidx).astype(jnp.float32)
    a_src_t = jax.lax.dot_general(mh * srcv, xl, (((1,), (1,)), ((), ())),
                                  preferred_element_type=jnp.float32)
    a_dst_t = jax.lax.dot_general(mh * dstv, xl, (((1,), (1,)), ((), ())),
                                  preferred_element_type=jnp.float32)
    ones_col = jnp.ones((n, 1), jnp.float32)
    rows = []
    for r in range(_ROWS_PER):
        sl = slice(r * n, (r + 1) * n)
        asr = a_src_t[:, sl]
        adr = a_dst_t[:, sl]
        xlr = xl[sl, :]
        m1 = jnp.max(asr, axis=1, keepdims=True)
        mx = adr + m1
        mx = jnp.where(mx >= 0, mx, 0.2 * mx)
        s = adr[:, :, None] + asr[:, None, :]
        s = jnp.where(s >= 0, s, 0.2 * s)
        ex = jnp.exp(s - mx[:, :, None]).reshape(hds * n, n)
        u = jax.lax.dot_general(ex, xlr, (((1,), (0,)), ((), ())),
                                preferred_element_type=jnp.float32)
        den = jax.lax.dot_general(ex, ones_col, (((1,), (0,)), ((), ())),
                                  preferred_element_type=jnp.float32)
        u = u / (den + 1e-16)
        out_r = bias * jnp.ones((n, 1), jnp.float32)
        for h in range(hds):
            out_r = out_r + u[h * n:(h + 1) * n, :] * mh[h:h + 1, :]
        rows.append(out_r)
    return jnp.concatenate(rows, axis=0)


def _gnn_kernel(x_ref, w0, s0, d0, b0, w1, s1, d1, b1, w2, s2, d2, b2,
                lw_ref, out_ref):
    x = x_ref[0].reshape(_NB, 16)
    x = _gat_layer(x, w0[0], s0[0], d0[0], b0[0], 8, 16)
    x = _gat_layer(x, w1[0], s1[0], d1[0], b1[0], 8, 16)
    x = _gat_layer(x, w2[0], s2[0], d2[0], b2[0], 8, 8)
    ridx = jax.lax.broadcasted_iota(jnp.int32, (_ROWS_PER, _NB), 0)
    nidx = jax.lax.broadcasted_iota(jnp.int32, (_ROWS_PER, _NB), 1)
    pm = (nidx // _NUM_COLS == ridx).astype(jnp.float32)
    cols = jax.lax.dot_general(pm, x, (((1,), (0,)), ((), ())),
                               preferred_element_type=jnp.float32)
    pred = jax.lax.dot_general(cols, lw_ref[0], (((1,), (1,)), ((), ())),
                               preferred_element_type=jnp.float32)
    out_ref[0] = pred


def _bcast_spec(shape):
    nd = len(shape)
    return pl.BlockSpec((1,) + shape[1:],
                        lambda b, r, _nd=nd: (b,) + (0,) * (_nd - 1))


@jax.jit
def kernel(xs_meta, pairs_meta, params):
    embed, pos_enc = _d2v_batch(pairs_meta, params)
    gat_w, lin_w = _weight_gen_batch(embed, params)
    (w0, s0, d0, b0), (w1, s1, d1, b1), (w2, s2, d2, b2) = gat_w

    xs3 = xs_meta.reshape(_BS, _NUM_ROWS * _NUM_COLS, 1)
    pe = jnp.broadcast_to(pos_enc[:, None, :, :],
                          (_BS, _NUM_ROWS, _NUM_COLS, _POS_ENC_DIM))
    pe = pe.reshape(_BS, _NUM_ROWS * _NUM_COLS, _POS_ENC_DIM)
    x_all = jnp.concatenate([xs3, pe], axis=-1)

    grid = (_BS, _NUM_ROWS // _ROWS_PER)
    out = pl.pallas_call(
        _gnn_kernel,
        grid=grid,
        in_specs=[
            pl.BlockSpec((1, _NB, 16), lambda b, g: (b, g, 0)),
            _bcast_spec(w0.shape), _bcast_spec(s0.shape),
            _bcast_spec(d0.shape), _bcast_spec(b0.shape),
            _bcast_spec(w1.shape), _bcast_spec(s1.shape),
            _bcast_spec(d1.shape), _bcast_spec(b1.shape),
            _bcast_spec(w2.shape), _bcast_spec(s2.shape),
            _bcast_spec(d2.shape), _bcast_spec(b2.shape),
            _bcast_spec(lin_w.shape),
        ],
        out_specs=pl.BlockSpec((1, _ROWS_PER, _NUM_CLASSES),
                               lambda b, g: (b, g, 0)),
        out_shape=jax.ShapeDtypeStruct((_BS, _NUM_ROWS, _NUM_CLASSES),
                                       jnp.float32),
    )(x_all, w0, s0, d0, b0, w1, s1, d1, b1, w2, s2, d2, b2, lin_w)
    return out


# speedup vs baseline: 1998.6846x; 1.4957x over previous
import jax
import jax.numpy as jnp
from jax.experimental import pallas as pl

_BS = 8
_NUM_ROWS = 64
_NUM_COLS = 64
_POS_ENC_DIM = 15
_OUT_SIZES = [(16, 128, 8), (128, 128, 8), (128, 64, 8)]
_NUM_CLASSES = 2
_GAT_OUT = 64
_ROWS_PER = 16
_NB = _ROWS_PER * _NUM_COLS


def _lin(x, w, b):
    return x @ w.T + b


def _d2v_batch(pairs_meta, p):
    x = jax.nn.relu(_lin(pairs_meta, p["f1_w"], p["f1_b"]))
    x = x + jax.nn.relu(_lin(x, p["f2r_w"], p["f2r_b"]))
    x = jax.nn.relu(_lin(x, p["f5_w"], p["f5_b"]))
    x = jnp.mean(x, axis=1)
    x_save = x
    x = jax.nn.relu(_lin(x, p["g1_w"], p["g1_b"]))
    x = jax.nn.relu(_lin(x, p["g2_w"], p["g2_b"]))
    x = jnp.mean(x, axis=1)
    x = jax.nn.relu(_lin(x, p["h1_w"], p["h1_b"]))
    x = x + jax.nn.relu(_lin(x, p["h2r_w"], p["h2r_b"]))
    x = jax.nn.relu(_lin(x, p["h5_w"], p["h5_b"]))
    pos = _lin(jax.nn.relu(_lin(x_save, p["p1_w"], p["p1_b"])),
               p["p2_w"], p["p2_b"])
    return x, pos


def _norm_rows(v):
    n = jnp.sqrt(jnp.sum(v * v, axis=-1, keepdims=True))
    return v / jnp.maximum(n, 1e-12)


def _weight_gen_batch(embed, p):
    out = []
    for i, (gin, godim, hds) in enumerate(_OUT_SIZES):
        gout = godim // hds
        all_w = _lin(jax.nn.relu(_lin(embed, p[f"wg{i}_w1"], p[f"wg{i}_b1"])),
                     p[f"wg{i}_w2"], p[f"wg{i}_b2"])
        s0 = gout * hds * gin
        s1 = hds * gout
        lin_w = _norm_rows(all_w[:, :s0]).reshape(_BS, gout * hds, gin)
        src_w = _norm_rows(all_w[:, s0:s0 + s1]).reshape(_BS, 1, hds * gout)
        dst_w = _norm_rows(all_w[:, s0 + s1:s0 + 2 * s1]).reshape(_BS, 1, hds * gout)
        bias_w = _norm_rows(all_w[:, s0 + 2 * s1:]).reshape(_BS, 1, gout * hds)
        out.append((lin_w, src_w, dst_w, bias_w))
    lw = _lin(jax.nn.relu(_lin(embed, p["wgl_w1"], p["wgl_b1"])),
              p["wgl_w2"], p["wgl_b2"]).reshape(_BS, _NUM_CLASSES, _GAT_OUT)
    return out, lw


def _gat_layer(x, w, srcv, dstv, bias, hds, gout):
    godim = hds * gout
    n = _NUM_COLS
    xl = jax.lax.dot_general(x, w, (((1,), (1,)), ((), ())),
                             preferred_element_type=jnp.float32)
    cidx = jax.lax.broadcasted_iota(jnp.int32, (hds, godim), 1)
    hidx = jax.lax.broadcasted_iota(jnp.int32, (hds, godim), 0)
    mh = (cidx // gout == hidx).astype(jnp.float32)
    a_src_t = jax.lax.dot_general(mh * srcv, xl, (((1,), (1,)), ((), ())),
                                  preferred_element_type=jnp.float32)
    a_dst_t = jax.lax.dot_general(mh * dstv, xl, (((1,), (1,)), ((), ())),
                                  preferred_element_type=jnp.float32)
    a_src64 = jax.lax.dot_general(xl * srcv, mh, (((1,), (1,)), ((), ())),
                                  preferred_element_type=jnp.float32)
    hn = hds * n
    lidx = jax.lax.broadcasted_iota(jnp.int32, (hds, hn), 1)
    hidx2 = jax.lax.broadcasted_iota(jnp.int32, (hds, hn), 0)
    rt = (lidx // n == hidx2).astype(jnp.float32)
    rows = []
    for r in range(_ROWS_PER):
        sl = slice(r * n, (r + 1) * n)
        asr = a_src_t[:, sl]
        adr = a_dst_t[:, sl]
        xlr = xl[sl, :]
        m1 = jnp.max(asr, axis=1, keepdims=True)
        mx8 = adr + m1
        mx8 = jnp.where(mx8 >= 0, mx8, 0.2 * mx8)
        ad_row = jnp.sum(rt * jnp.tile(adr, (1, hds)), axis=0, keepdims=True)
        mx_row = jnp.sum(rt * jnp.tile(mx8, (1, hds)), axis=0, keepdims=True)
        s = jax.lax.dot_general(a_src64[sl, :], rt, (((1,), (0,)), ((), ())),
                                preferred_element_type=jnp.float32)
        s = s + ad_row
        s = jnp.where(s >= 0, s, 0.2 * s)
        ex = jnp.exp(s - mx_row)
        den = jnp.sum(ex, axis=0, keepdims=True)
        ex = ex * (1.0 / (den + 1e-16))
        u = jax.lax.dot_general(ex, xlr, (((0,), (0,)), ((), ())),
                                preferred_element_type=jnp.float32)
        parts = [u[h * n:(h + 1) * n, :] * mh[h:h + 1, :] for h in range(hds)]
        while len(parts) > 1:
            parts = [parts[i] + parts[i + 1] for i in range(0, len(parts), 2)]
        rows.append(parts[0] + bias)
    return jnp.concatenate(rows, axis=0)


def _gnn_kernel(x_ref, w0, s0, d0, b0, w1, s1, d1, b1, w2, s2, d2, b2,
                lw_ref, out_ref):
    x = x_ref[0].reshape(_NB, 16)
    x = _gat_layer(x, w0[0], s0[0], d0[0], b0[0], 8, 16)
    x = _gat_layer(x, w1[0], s1[0], d1[0], b1[0], 8, 16)
    x = _gat_layer(x, w2[0], s2[0], d2[0], b2[0], 8, 8)
    ridx = jax.lax.broadcasted_iota(jnp.int32, (_ROWS_PER, _NB), 0)
    nidx = jax.lax.broadcasted_iota(jnp.int32, (_ROWS_PER, _NB), 1)
    pm = (nidx // _NUM_COLS == ridx).astype(jnp.float32)
    cols = jax.lax.dot_general(pm, x, (((1,), (0,)), ((), ())),
                               preferred_element_type=jnp.float32)
    pred = jax.lax.dot_general(cols, lw_ref[0], (((1,), (1,)), ((), ())),
                               preferred_element_type=jnp.float32)
    out_ref[0] = pred


def _bcast_spec(shape):
    nd = len(shape)
    return pl.BlockSpec((1,) + shape[1:],
                        lambda b, r, _nd=nd: (b,) + (0,) * (_nd - 1))


@jax.jit
def kernel(xs_meta, pairs_meta, params):
    embed, pos_enc = _d2v_batch(pairs_meta, params)
    gat_w, lin_w = _weight_gen_batch(embed, params)
    (w0, s0, d0, b0), (w1, s1, d1, b1), (w2, s2, d2, b2) = gat_w

    xs3 = xs_meta.reshape(_BS, _NUM_ROWS * _NUM_COLS, 1)
    pe = jnp.broadcast_to(pos_enc[:, None, :, :],
                          (_BS, _NUM_ROWS, _NUM_COLS, _POS_ENC_DIM))
    pe = pe.reshape(_BS, _NUM_ROWS * _NUM_COLS, _POS_ENC_DIM)
    x_all = jnp.concatenate([xs3, pe], axis=-1)

    grid = (_BS, _NUM_ROWS // _ROWS_PER)
    out = pl.pallas_call(
        _gnn_kernel,
        grid=grid,
        in_specs=[
            pl.BlockSpec((1, _NB, 16), lambda b, g: (b, g, 0)),
            _bcast_spec(w0.shape), _bcast_spec(s0.shape),
            _bcast_spec(d0.shape), _bcast_spec(b0.shape),
            _bcast_spec(w1.shape), _bcast_spec(s1.shape),
            _bcast_spec(d1.shape), _bcast_spec(b1.shape),
            _bcast_spec(w2.shape), _bcast_spec(s2.shape),
            _bcast_spec(d2.shape), _bcast_spec(b2.shape),
            _bcast_spec(lin_w.shape),
        ],
        out_specs=pl.BlockSpec((1, _ROWS_PER, _NUM_CLASSES),
                               lambda b, g: (b, g, 0)),
        out_shape=jax.ShapeDtypeStruct((_BS, _NUM_ROWS, _NUM_CLASSES),
                                       jnp.float32),
    )(x_all, w0, s0, d0, b0, w1, s1, d1, b1, w2, s2, d2, b2, lin_w)
    return out


# speedup vs baseline: 2146.1202x; 1.0738x over previous
import jax
import jax.numpy as jnp
from jax.experimental import pallas as pl

_BS = 8
_NUM_ROWS = 64
_NUM_COLS = 64
_POS_ENC_DIM = 15
_OUT_SIZES = [(16, 128, 8), (128, 128, 8), (128, 64, 8)]
_NUM_CLASSES = 2
_GAT_OUT = 64
_ROWS_PER = 32
_NB = _ROWS_PER * _NUM_COLS


def _lin(x, w, b):
    return x @ w.T + b


def _d2v_batch(pairs_meta, p):
    x = jax.nn.relu(_lin(pairs_meta, p["f1_w"], p["f1_b"]))
    x = x + jax.nn.relu(_lin(x, p["f2r_w"], p["f2r_b"]))
    x = jax.nn.relu(_lin(x, p["f5_w"], p["f5_b"]))
    x = jnp.mean(x, axis=1)
    x_save = x
    x = jax.nn.relu(_lin(x, p["g1_w"], p["g1_b"]))
    x = jax.nn.relu(_lin(x, p["g2_w"], p["g2_b"]))
    x = jnp.mean(x, axis=1)
    x = jax.nn.relu(_lin(x, p["h1_w"], p["h1_b"]))
    x = x + jax.nn.relu(_lin(x, p["h2r_w"], p["h2r_b"]))
    x = jax.nn.relu(_lin(x, p["h5_w"], p["h5_b"]))
    pos = _lin(jax.nn.relu(_lin(x_save, p["p1_w"], p["p1_b"])),
               p["p2_w"], p["p2_b"])
    return x, pos


def _norm_rows(v):
    n = jnp.sqrt(jnp.sum(v * v, axis=-1, keepdims=True))
    return v / jnp.maximum(n, 1e-12)


def _weight_gen_batch(embed, p):
    out = []
    for i, (gin, godim, hds) in enumerate(_OUT_SIZES):
        gout = godim // hds
        all_w = _lin(jax.nn.relu(_lin(embed, p[f"wg{i}_w1"], p[f"wg{i}_b1"])),
                     p[f"wg{i}_w2"], p[f"wg{i}_b2"])
        s0 = gout * hds * gin
        s1 = hds * gout
        lin_w = _norm_rows(all_w[:, :s0]).reshape(_BS, gout * hds, gin)
        src_w = _norm_rows(all_w[:, s0:s0 + s1]).reshape(_BS, 1, hds * gout)
        dst_w = _norm_rows(all_w[:, s0 + s1:s0 + 2 * s1]).reshape(_BS, 1, hds * gout)
        bias_w = _norm_rows(all_w[:, s0 + 2 * s1:]).reshape(_BS, 1, gout * hds)
        out.append((lin_w, src_w, dst_w, bias_w))
    lw = _lin(jax.nn.relu(_lin(embed, p["wgl_w1"], p["wgl_b1"])),
              p["wgl_w2"], p["wgl_b2"]).reshape(_BS, _NUM_CLASSES, _GAT_OUT)
    return out, lw


def _gat_layer(x, w, srcv, dstv, bias, hds, gout):
    godim = hds * gout
    n = _NUM_COLS
    xl = jax.lax.dot_general(x, w, (((1,), (1,)), ((), ())),
                             preferred_element_type=jnp.float32)
    cidx = jax.lax.broadcasted_iota(jnp.int32, (hds, godim), 1)
    hidx = jax.lax.broadcasted_iota(jnp.int32, (hds, godim), 0)
    mh = (cidx // gout == hidx).astype(jnp.float32)
    msrc = mh * srcv
    a_src_t = jax.lax.dot_general(msrc, xl, (((1,), (1,)), ((), ())),
                                  preferred_element_type=jnp.float32)
    a_dst_t = jax.lax.dot_general(mh * dstv, xl, (((1,), (1,)), ((), ())),
                                  preferred_element_type=jnp.float32)
    a_src64 = jax.lax.dot_general(xl, msrc, (((1,), (1,)), ((), ())),
                                  preferred_element_type=jnp.float32)
    ones_row = jnp.ones((1, n), jnp.float32)
    hn = hds * n
    lidx = jax.lax.broadcasted_iota(jnp.int32, (hds, hn), 1)
    hidx2 = jax.lax.broadcasted_iota(jnp.int32, (hds, hn), 0)
    rt = (lidx // n == hidx2).astype(jnp.float32)
    rows = []
    for r in range(_ROWS_PER):
        sl = slice(r * n, (r + 1) * n)
        asr = a_src_t[:, sl]
        adr = a_dst_t[:, sl]
        xlr = xl[sl, :]
        m1 = jnp.max(asr, axis=1, keepdims=True)
        mx8 = adr + m1
        mx8 = jnp.where(mx8 >= 0, mx8, 0.2 * mx8)
        ad_row = jnp.sum(rt * jnp.tile(adr, (1, hds)), axis=0, keepdims=True)
        mx_row = jnp.sum(rt * jnp.tile(mx8, (1, hds)), axis=0, keepdims=True)
        s = jax.lax.dot_general(a_src64[sl, :], rt, (((1,), (0,)), ((), ())),
                                preferred_element_type=jnp.float32)
        s = s + ad_row
        s = jnp.where(s >= 0, s, 0.2 * s)
        ex = jnp.exp(s - mx_row)
        den = jnp.sum(ex, axis=0, keepdims=True)
        ex = ex * (1.0 / (den + 1e-16))
        u = jax.lax.dot_general(ex, xlr, (((0,), (0,)), ((), ())),
                                preferred_element_type=jnp.float32)
        parts = [u[h * n:(h + 1) * n, :] * mh[h:h + 1, :] for h in range(hds)]
        while len(parts) > 1:
            parts = [parts[i] + parts[i + 1] for i in range(0, len(parts), 2)]
        rows.append(parts[0] + bias)
    return jnp.concatenate(rows, axis=0)


def _gnn_kernel(x_ref, w0, s0, d0, b0, w1, s1, d1, b1, w2, s2, d2, b2,
                lw_ref, out_ref):
    x = x_ref[0].reshape(_NB, 16)
    x = _gat_layer(x, w0[0], s0[0], d0[0], b0[0], 8, 16)
    x = _gat_layer(x, w1[0], s1[0], d1[0], b1[0], 8, 16)
    x = _gat_layer(x, w2[0], s2[0], d2[0], b2[0], 8, 8)
    ridx = jax.lax.broadcasted_iota(jnp.int32, (_ROWS_PER, _NB), 0)
    nidx = jax.lax.broadcasted_iota(jnp.int32, (_ROWS_PER, _NB), 1)
    pm = (nidx // _NUM_COLS == ridx).astype(jnp.float32)
    cols = jax.lax.dot_general(pm, x, (((1,), (0,)), ((), ())),
                               preferred_element_type=jnp.float32)
    pred = jax.lax.dot_general(cols, lw_ref[0], (((1,), (1,)), ((), ())),
                               preferred_element_type=jnp.float32)
    out_ref[0] = pred


def _bcast_spec(shape):
    nd = len(shape)
    return pl.BlockSpec((1,) + shape[1:],
                        lambda b, r, _nd=nd: (b,) + (0,) * (_nd - 1))


@jax.jit
def kernel(xs_meta, pairs_meta, params):
    embed, pos_enc = _d2v_batch(pairs_meta, params)
    gat_w, lin_w = _weight_gen_batch(embed, params)
    (w0, s0, d0, b0), (w1, s1, d1, b1), (w2, s2, d2, b2) = gat_w

    xs3 = xs_meta.reshape(_BS, _NUM_ROWS * _NUM_COLS, 1)
    pe = jnp.broadcast_to(pos_enc[:, None, :, :],
                          (_BS, _NUM_ROWS, _NUM_COLS, _POS_ENC_DIM))
    pe = pe.reshape(_BS, _NUM_ROWS * _NUM_COLS, _POS_ENC_DIM)
    x_all = jnp.concatenate([xs3, pe], axis=-1)

    grid = (_BS, _NUM_ROWS // _ROWS_PER)
    out = pl.pallas_call(
        _gnn_kernel,
        grid=grid,
        in_specs=[
            pl.BlockSpec((1, _NB, 16), lambda b, g: (b, g, 0)),
            _bcast_spec(w0.shape), _bcast_spec(s0.shape),
            _bcast_spec(d0.shape), _bcast_spec(b0.shape),
            _bcast_spec(w1.shape), _bcast_spec(s1.shape),
            _bcast_spec(d1.shape), _bcast_spec(b1.shape),
            _bcast_spec(w2.shape), _bcast_spec(s2.shape),
            _bcast_spec(d2.shape), _bcast_spec(b2.shape),
            _bcast_spec(lin_w.shape),
        ],
        out_specs=pl.BlockSpec((1, _ROWS_PER, _NUM_CLASSES),
                               lambda b, g: (b, g, 0)),
        out_shape=jax.ShapeDtypeStruct((_BS, _NUM_ROWS, _NUM_CLASSES),
                                       jnp.float32),
    )(x_all, w0, s0, d0, b0, w1, s1, d1, b1, w2, s2, d2, b2, lin_w)
    return out


# speedup vs baseline: 2149.2759x; 1.0015x over previous
import jax
import jax.numpy as jnp
from jax.experimental import pallas as pl
from jax.experimental.pallas import tpu as pltpu

_BS = 8
_NUM_ROWS = 64
_NUM_COLS = 64
_POS_ENC_DIM = 15
_OUT_SIZES = [(16, 128, 8), (128, 128, 8), (128, 64, 8)]
_NUM_CLASSES = 2
_GAT_OUT = 64
_ROWS_PER = 32
_NB = _ROWS_PER * _NUM_COLS


def _lin(x, w, b):
    return x @ w.T + b


def _d2v_batch(pairs_meta, p):
    x = jax.nn.relu(_lin(pairs_meta, p["f1_w"], p["f1_b"]))
    x = x + jax.nn.relu(_lin(x, p["f2r_w"], p["f2r_b"]))
    x = jax.nn.relu(_lin(x, p["f5_w"], p["f5_b"]))
    x = jnp.mean(x, axis=1)
    x_save = x
    x = jax.nn.relu(_lin(x, p["g1_w"], p["g1_b"]))
    x = jax.nn.relu(_lin(x, p["g2_w"], p["g2_b"]))
    x = jnp.mean(x, axis=1)
    x = jax.nn.relu(_lin(x, p["h1_w"], p["h1_b"]))
    x = x + jax.nn.relu(_lin(x, p["h2r_w"], p["h2r_b"]))
    x = jax.nn.relu(_lin(x, p["h5_w"], p["h5_b"]))
    pos = _lin(jax.nn.relu(_lin(x_save, p["p1_w"], p["p1_b"])),
               p["p2_w"], p["p2_b"])
    return x, pos


def _norm_rows(v):
    n = jnp.sqrt(jnp.sum(v * v, axis=-1, keepdims=True))
    return v / jnp.maximum(n, 1e-12)


def _weight_gen_batch(embed, p):
    out = []
    for i, (gin, godim, hds) in enumerate(_OUT_SIZES):
        gout = godim // hds
        all_w = _lin(jax.nn.relu(_lin(embed, p[f"wg{i}_w1"], p[f"wg{i}_b1"])),
                     p[f"wg{i}_w2"], p[f"wg{i}_b2"])
        s0 = gout * hds * gin
        s1 = hds * gout
        lin_w = _norm_rows(all_w[:, :s0]).reshape(_BS, gout * hds, gin)
        src_w = _norm_rows(all_w[:, s0:s0 + s1]).reshape(_BS, 1, hds * gout)
        dst_w = _norm_rows(all_w[:, s0 + s1:s0 + 2 * s1]).reshape(_BS, 1, hds * gout)
        bias_w = _norm_rows(all_w[:, s0 + 2 * s1:]).reshape(_BS, 1, gout * hds)
        out.append((lin_w, src_w, dst_w, bias_w))
    lw = _lin(jax.nn.relu(_lin(embed, p["wgl_w1"], p["wgl_b1"])),
              p["wgl_w2"], p["wgl_b2"]).reshape(_BS, _NUM_CLASSES, _GAT_OUT)
    return out, lw


def _gat_layer(x, w, srcv, dstv, bias, hds, gout):
    godim = hds * gout
    n = _NUM_COLS
    xl = jax.lax.dot_general(x, w, (((1,), (1,)), ((), ())),
                             preferred_element_type=jnp.float32)
    cidx = jax.lax.broadcasted_iota(jnp.int32, (hds, godim), 1)
    hidx = jax.lax.broadcasted_iota(jnp.int32, (hds, godim), 0)
    mh = (cidx // gout == hidx).astype(jnp.float32)
    msrc = mh * srcv
    a_src_t = jax.lax.dot_general(msrc, xl, (((1,), (1,)), ((), ())),
                                  preferred_element_type=jnp.float32)
    a_dst_t = jax.lax.dot_general(mh * dstv, xl, (((1,), (1,)), ((), ())),
                                  preferred_element_type=jnp.float32)
    a_src64 = jax.lax.dot_general(xl, msrc, (((1,), (1,)), ((), ())),
                                  preferred_element_type=jnp.float32)
    ones_row = jnp.ones((1, n), jnp.float32)
    hn = hds * n
    lidx = jax.lax.broadcasted_iota(jnp.int32, (hds, hn), 1)
    hidx2 = jax.lax.broadcasted_iota(jnp.int32, (hds, hn), 0)
    rt = (lidx // n == hidx2).astype(jnp.float32)
    rows = []
    for r in range(_ROWS_PER):
        sl = slice(r * n, (r + 1) * n)
        asr = a_src_t[:, sl]
        adr = a_dst_t[:, sl]
        xlr = xl[sl, :]
        m1 = jnp.max(asr, axis=1, keepdims=True)
        mx8 = adr + m1
        mx8 = jnp.where(mx8 >= 0, mx8, 0.2 * mx8)
        ad_row = jnp.sum(rt * jnp.tile(adr, (1, hds)), axis=0, keepdims=True)
        mx_row = jnp.sum(rt * jnp.tile(mx8, (1, hds)), axis=0, keepdims=True)
        s = jax.lax.dot_general(a_src64[sl, :], rt, (((1,), (0,)), ((), ())),
                                preferred_element_type=jnp.float32)
        s = s + ad_row
        s = jnp.where(s >= 0, s, 0.2 * s)
        ex = jnp.exp(s - mx_row)
        den = jnp.sum(ex, axis=0, keepdims=True)
        ex = ex * (1.0 / (den + 1e-16))
        u = jax.lax.dot_general(ex, xlr, (((0,), (0,)), ((), ())),
                                preferred_element_type=jnp.float32)
        parts = [u[h * n:(h + 1) * n, :] * mh[h:h + 1, :] for h in range(hds)]
        while len(parts) > 1:
            parts = [parts[i] + parts[i + 1] for i in range(0, len(parts), 2)]
        rows.append(parts[0] + bias)
    return jnp.concatenate(rows, axis=0)


def _gnn_kernel(x_ref, w0, s0, d0, b0, w1, s1, d1, b1, w2, s2, d2, b2,
                lw_ref, out_ref):
    x = x_ref[0].reshape(_NB, 16)
    x = _gat_layer(x, w0[0], s0[0], d0[0], b0[0], 8, 16)
    x = _gat_layer(x, w1[0], s1[0], d1[0], b1[0], 8, 16)
    x = _gat_layer(x, w2[0], s2[0], d2[0], b2[0], 8, 8)
    ridx = jax.lax.broadcasted_iota(jnp.int32, (_ROWS_PER, _NB), 0)
    nidx = jax.lax.broadcasted_iota(jnp.int32, (_ROWS_PER, _NB), 1)
    pm = (nidx // _NUM_COLS == ridx).astype(jnp.float32)
    cols = jax.lax.dot_general(pm, x, (((1,), (0,)), ((), ())),
                               preferred_element_type=jnp.float32)
    pred = jax.lax.dot_general(cols, lw_ref[0], (((1,), (1,)), ((), ())),
                               preferred_element_type=jnp.float32)
    out_ref[0] = pred


def _bcast_spec(shape):
    nd = len(shape)
    return pl.BlockSpec((1,) + shape[1:],
                        lambda b, r, _nd=nd: (b,) + (0,) * (_nd - 1))


@jax.jit
def kernel(xs_meta, pairs_meta, params):
    embed, pos_enc = _d2v_batch(pairs_meta, params)
    gat_w, lin_w = _weight_gen_batch(embed, params)
    (w0, s0, d0, b0), (w1, s1, d1, b1), (w2, s2, d2, b2) = gat_w

    xs3 = xs_meta.reshape(_BS, _NUM_ROWS * _NUM_COLS, 1)
    pe = jnp.broadcast_to(pos_enc[:, None, :, :],
                          (_BS, _NUM_ROWS, _NUM_COLS, _POS_ENC_DIM))
    pe = pe.reshape(_BS, _NUM_ROWS * _NUM_COLS, _POS_ENC_DIM)
    x_all = jnp.concatenate([xs3, pe], axis=-1)

    grid = (_BS, _NUM_ROWS // _ROWS_PER)
    out = pl.pallas_call(
        _gnn_kernel,
        grid=grid,
        in_specs=[
            pl.BlockSpec((1, _NB, 16), lambda b, g: (b, g, 0)),
            _bcast_spec(w0.shape), _bcast_spec(s0.shape),
            _bcast_spec(d0.shape), _bcast_spec(b0.shape),
            _bcast_spec(w1.shape), _bcast_spec(s1.shape),
            _bcast_spec(d1.shape), _bcast_spec(b1.shape),
            _bcast_spec(w2.shape), _bcast_spec(s2.shape),
            _bcast_spec(d2.shape), _bcast_spec(b2.shape),
            _bcast_spec(lin_w.shape),
        ],
        out_specs=pl.BlockSpec((1, _ROWS_PER, _NUM_CLASSES),
                               lambda b, g: (b, g, 0)),
        out_shape=jax.ShapeDtypeStruct((_BS, _NUM_ROWS, _NUM_CLASSES),
                                       jnp.float32),
        compiler_params=pltpu.CompilerParams(
            dimension_semantics=("parallel", "parallel")),
    )(x_all, w0, s0, d0, b0, w1, s1, d1, b1, w2, s2, d2, b2, lin_w)
    return out
